# baseline (speedup 1.0000x reference)
import functools

import jax
import jax.numpy as jnp
from jax import lax
from jax.experimental import pallas as pl
from jax.experimental.pallas import tpu as pltpu
from jax.experimental.pallas import tpu_sc as plsc

N_USERS = 30000
N_ITEMS = 20000
N = N_USERS + N_ITEMS
NNZ = 800000
D = 64
DQ = D // 4

NC = 2
NS = 16

ROW_BLK = 512
N_PAD = 50176
STRIPE = N_PAD // NS

CHUNK = 1024
GRP = 128
NGRP = CHUNK // GRP
NCHUNK = 50
SUPER = 10
NSUPER = NCHUNK // SUPER
E_TILE = CHUNK * NCHUNK
NNZ_PAD = E_TILE * NS


def _spmm_body(q0, q1, q2, q3, row2d, col2d, val, zrows,
               s0, s1, s2, s3,
               acc, colbuf, rowbuf, valbuf, gbuf0, gbuf1, gsem, ssem):
  c = lax.axis_index("c")
  s = lax.axis_index("s")
  stripe0 = s * STRIPE
  bufs = (gbuf0, gbuf1)

  def scale_chunk(buf, ci):
    def grp_body(gi, carry2):
      vals = valbuf[pl.ds(ci * CHUNK + gi * 16, 16)]
      e0 = gi * 16
      for l in range(16):
        v = jnp.take_along_axis(vals, jnp.full((16,), l, jnp.int32), axis=0)
        e = e0 + l
        buf[e, pl.ds(0, 16)] = buf[e, pl.ds(0, 16)] * v
      return carry2

    lax.fori_loop(0, CHUNK // 16, grp_body, 0)

  def do_pass(ego_hbm, out_hbm):
    pltpu.sync_copy(zrows.at[pl.ds(stripe0, STRIPE)],
                    acc.at[pl.ds(stripe0, STRIPE)])
    plsc.subcore_barrier()

    def super_body(si, carry):
      off128 = (s * NCHUNK + si * SUPER) * NGRP
      offe = (s * NCHUNK + si * SUPER) * CHUNK
      pltpu.sync_copy(col2d.at[pl.ds(off128, SUPER * NGRP)], colbuf)
      pltpu.sync_copy(row2d.at[pl.ds(off128, SUPER * NGRP)], rowbuf)
      pltpu.sync_copy(val.at[pl.ds(offe, SUPER * CHUNK)], valbuf)

      def gather(ci):
        buf = bufs[ci % 2]
        return [
            pltpu.async_copy(ego_hbm.at[colbuf.at[ci * NGRP + g]],
                             buf.at[pl.ds(g * GRP, GRP)], gsem)
            for g in range(NGRP)
        ]

      def scatter(ci):
        buf = bufs[ci % 2]
        return [
            pltpu.async_copy(buf.at[pl.ds(g * GRP, GRP)],
                             acc.at[rowbuf.at[ci * NGRP + g]], ssem, add=True)
            for g in range(NGRP)
        ]

      gd = gather(0)
      sd = [None] * SUPER
      for ci in range(SUPER):
        for d in gd:
          d.wait()
        scale_chunk(bufs[ci % 2], ci)
        sd[ci] = scatter(ci)
        if ci + 1 < SUPER:
          if ci >= 1:
            for d in sd[ci - 1]:
              d.wait()
          gd = gather(ci + 1)
      for d in sd[SUPER - 2] + sd[SUPER - 1]:
        d.wait()
      return carry

    lax.fori_loop(0, NSUPER, super_body, 0)
    plsc.subcore_barrier()
    pltpu.sync_copy(acc.at[pl.ds(stripe0, STRIPE)],
                    out_hbm.at[pl.ds(stripe0, STRIPE)])
    plsc.subcore_barrier()

  @pl.when(c == 0)
  def _():
    do_pass(q0, s0)
    do_pass(q2, s2)

  @pl.when(c == 1)
  def _():
    do_pass(q1, s1)
    do_pass(q3, s3)


_spmm = pl.kernel(
    _spmm_body,
    out_type=tuple(
        jax.ShapeDtypeStruct((N_PAD, DQ), jnp.float32) for _ in range(4)
    ),
    mesh=plsc.VectorSubcoreMesh(core_axis_name="c", subcore_axis_name="s",
                                num_cores=NC, num_subcores=NS),
    compiler_params=pltpu.CompilerParams(use_tc_tiling_on_sc=False),
    scratch_types=[
        pltpu.VMEM_SHARED((N_PAD, DQ), jnp.float32),
        pltpu.VMEM((SUPER * NGRP, GRP), jnp.int32),
        pltpu.VMEM((SUPER * NGRP, GRP), jnp.int32),
        pltpu.VMEM((SUPER * CHUNK,), jnp.float32),
        pltpu.VMEM((CHUNK, DQ), jnp.float32),
        pltpu.VMEM((CHUNK, DQ), jnp.float32),
        pltpu.SemaphoreType.DMA,
        pltpu.SemaphoreType.DMA,
    ],
)


def _hop_body(is_last, e0_ref, e1_ref, e2_ref, e3_ref,
              s0_ref, s1_ref, s2_ref, s3_ref,
              wgc_ref, bgc_ref, wbi_ref, bbi_ref, accin_ref, *outs):
  ego = jnp.concatenate(
      [e0_ref[...], e1_ref[...], e2_ref[...], e3_ref[...]], axis=1)
  side = jnp.concatenate(
      [s0_ref[...], s1_ref[...], s2_ref[...], s3_ref[...]], axis=1)
  sum_e = jnp.dot(side, wgc_ref[...], precision=lax.Precision.HIGHEST,
                  preferred_element_type=jnp.float32) + bgc_ref[...]
  bi = jnp.dot(ego * side, wbi_ref[...], precision=lax.Precision.HIGHEST,
               preferred_element_type=jnp.float32) + bbi_ref[...]
  e = sum_e + bi
  e = jnp.where(e >= 0.0, e, 0.2 * e)
  nrm = jnp.maximum(jnp.sqrt(jnp.sum(e * e, axis=1, keepdims=True)), 1e-12)
  n = e / nrm
  if is_last:
    outs[0][...] = (accin_ref[...] + n) * 0.25
  else:
    for q in range(4):
      outs[q][...] = e[:, q * DQ:(q + 1) * DQ]
    outs[4][...] = accin_ref[...] + n


def _make_hop(is_last):
  nblk = N_PAD // ROW_BLK
  row = lambda i: (i, 0)
  full = lambda i: (0, 0)
  in_specs = (
      [pl.BlockSpec((ROW_BLK, DQ), row) for _ in range(8)]
      + [
          pl.BlockSpec((D, D), full),
          pl.BlockSpec((1, D), full),
          pl.BlockSpec((D, D), full),
          pl.BlockSpec((1, D), full),
          pl.BlockSpec((ROW_BLK, D), row),
      ]
  )
  if is_last:
    out_specs = [pl.BlockSpec((ROW_BLK, D), row)]
    out_shape = [jax.ShapeDtypeStruct((N_PAD, D), jnp.float32)]
  else:
    out_specs = (
        [pl.BlockSpec((ROW_BLK, DQ), row) for _ in range(4)]
        + [pl.BlockSpec((ROW_BLK, D), row)]
    )
    out_shape = (
        [jax.ShapeDtypeStruct((N_PAD, DQ), jnp.float32) for _ in range(4)]
        + [jax.ShapeDtypeStruct((N_PAD, D), jnp.float32)]
    )
  return pl.pallas_call(
      functools.partial(_hop_body, is_last),
      grid=(nblk,),
      in_specs=in_specs,
      out_specs=out_specs,
      out_shape=out_shape,
  )


_hop_mid = _make_hop(False)
_hop_last = _make_hop(True)
HOPS_LAST = 2


@jax.jit
def kernel(user_emb, item_emb, adj_idx, adj_val,
           W_gc_0, b_gc_0, W_bi_0, b_bi_0,
           W_gc_1, b_gc_1, W_bi_1, b_bi_1,
           W_gc_2, b_gc_2, W_bi_2, b_bi_2):
  ego0 = jnp.concatenate([user_emb, item_emb], axis=0)
  ego0 = jnp.pad(ego0, ((0, N_PAD - N), (0, 0)))
  eq = [ego0[:, q * DQ:(q + 1) * DQ] for q in range(4)]

  row = adj_idx[0].astype(jnp.int32)
  col = adj_idx[1].astype(jnp.int32)
  rowp = jnp.pad(row, (0, NNZ_PAD - NNZ)).reshape(NNZ_PAD // GRP, GRP)
  colp = jnp.pad(col, (0, NNZ_PAD - NNZ)).reshape(NNZ_PAD // GRP, GRP)
  valp = jnp.pad(adj_val, (0, NNZ_PAD - NNZ))
  zrows = jnp.zeros((N_PAD, DQ), jnp.float32)

  weights = [(W_gc_0, b_gc_0, W_bi_0, b_bi_0),
             (W_gc_1, b_gc_1, W_bi_1, b_bi_1),
             (W_gc_2, b_gc_2, W_bi_2, b_bi_2)]

  acc = ego0
  for k, (wgc, bgc, wbi, bbi) in enumerate(weights):
    sq = _spmm(eq[0], eq[1], eq[2], eq[3], rowp, colp, valp, zrows)
    if k < HOPS_LAST:
      *eq, acc = _hop_mid(*eq, *sq, wgc, bgc, wbi, bbi, acc)
    else:
      final, = _hop_last(*eq, *sq, wgc, bgc, wbi, bbi, acc)
  return final[:N]


# speedup vs baseline: 1.2626x; 1.2626x over previous
import functools

import jax
import jax.numpy as jnp
from jax import lax
from jax.experimental import pallas as pl
from jax.experimental.pallas import tpu as pltpu
from jax.experimental.pallas import tpu_sc as plsc

N_USERS = 30000
N_ITEMS = 20000
N = N_USERS + N_ITEMS
NNZ = 800000
D = 64
DQ = D // 4

NC = 2
NS = 16

ROW_BLK = 512
N_PAD = 50176
STRIPE = N_PAD // NS

CHUNK = 512
GRP = 128
NGRP = CHUNK // GRP
NCHUNK = 100
SUPER = 5
NSUPER = NCHUNK // SUPER
E_TILE = CHUNK * NCHUNK
NNZ_PAD = E_TILE * NS


def _spmm_body(q0, q1, q2, q3, row2d, col2d, val, zrows,
               s0, s1, s2, s3,
               acc, tbl, colbuf, rowbuf, valbuf, gbuf0, gbuf1, gsem, ssem):
  c = lax.axis_index("c")
  s = lax.axis_index("s")
  stripe0 = s * STRIPE
  bufs = (gbuf0, gbuf1)

  def scale_chunk(buf, ci):
    def grp_body(gi, carry2):
      vals = valbuf[pl.ds(ci * CHUNK + gi * 16, 16)]
      e0 = gi * 16
      for l in range(16):
        v = jnp.take_along_axis(vals, jnp.full((16,), l, jnp.int32), axis=0)
        e = e0 + l
        buf[e, pl.ds(0, 16)] = buf[e, pl.ds(0, 16)] * v
      return carry2

    lax.fori_loop(0, CHUNK // 16, grp_body, 0)

  def do_pass(ego_hbm, out_hbm):
    pltpu.sync_copy(zrows.at[pl.ds(stripe0, STRIPE)],
                    acc.at[pl.ds(stripe0, STRIPE)])
    pltpu.sync_copy(ego_hbm.at[pl.ds(stripe0, STRIPE)],
                    tbl.at[pl.ds(stripe0, STRIPE)])
    plsc.subcore_barrier()

    def super_body(si, carry):
      off128 = (s * NCHUNK + si * SUPER) * NGRP
      offe = (s * NCHUNK + si * SUPER) * CHUNK
      pltpu.sync_copy(col2d.at[pl.ds(off128, SUPER * NGRP)], colbuf)
      pltpu.sync_copy(row2d.at[pl.ds(off128, SUPER * NGRP)], rowbuf)
      pltpu.sync_copy(val.at[pl.ds(offe, SUPER * CHUNK)], valbuf)

      def gather(ci):
        buf = bufs[ci % 2]
        return [
            pltpu.async_copy(tbl.at[colbuf.at[ci * NGRP + g]],
                             buf.at[pl.ds(g * GRP, GRP)], gsem)
            for g in range(NGRP)
        ]

      def scatter(ci):
        buf = bufs[ci % 2]
        return [
            pltpu.async_copy(buf.at[pl.ds(g * GRP, GRP)],
                             acc.at[rowbuf.at[ci * NGRP + g]], ssem, add=True)
            for g in range(NGRP)
        ]

      gd = gather(0)
      sd = [None] * SUPER
      for ci in range(SUPER):
        for d in gd:
          d.wait()
        scale_chunk(bufs[ci % 2], ci)
        sd[ci] = scatter(ci)
        if ci + 1 < SUPER:
          if ci >= 1:
            for d in sd[ci - 1]:
              d.wait()
          gd = gather(ci + 1)
      for d in sd[SUPER - 2] + sd[SUPER - 1]:
        d.wait()
      return carry

    lax.fori_loop(0, NSUPER, super_body, 0)
    plsc.subcore_barrier()
    pltpu.sync_copy(acc.at[pl.ds(stripe0, STRIPE)],
                    out_hbm.at[pl.ds(stripe0, STRIPE)])
    plsc.subcore_barrier()

  @pl.when(c == 0)
  def _():
    do_pass(q0, s0)
    do_pass(q2, s2)

  @pl.when(c == 1)
  def _():
    do_pass(q1, s1)
    do_pass(q3, s3)


_spmm = pl.kernel(
    _spmm_body,
    out_type=tuple(
        jax.ShapeDtypeStruct((N_PAD, DQ), jnp.float32) for _ in range(4)
    ),
    mesh=plsc.VectorSubcoreMesh(core_axis_name="c", subcore_axis_name="s",
                                num_cores=NC, num_subcores=NS),
    compiler_params=pltpu.CompilerParams(use_tc_tiling_on_sc=False),
    scratch_types=[
        pltpu.VMEM_SHARED((N_PAD, DQ), jnp.float32),
        pltpu.VMEM_SHARED((N_PAD, DQ), jnp.float32),
        pltpu.VMEM((SUPER * NGRP, GRP), jnp.int32),
        pltpu.VMEM((SUPER * NGRP, GRP), jnp.int32),
        pltpu.VMEM((SUPER * CHUNK,), jnp.float32),
        pltpu.VMEM((CHUNK, DQ), jnp.float32),
        pltpu.VMEM((CHUNK, DQ), jnp.float32),
        pltpu.SemaphoreType.DMA,
        pltpu.SemaphoreType.DMA,
    ],
)


def _hop_body(is_last, e0_ref, e1_ref, e2_ref, e3_ref,
              s0_ref, s1_ref, s2_ref, s3_ref,
              wgc_ref, bgc_ref, wbi_ref, bbi_ref, accin_ref, *outs):
  ego = jnp.concatenate(
      [e0_ref[...], e1_ref[...], e2_ref[...], e3_ref[...]], axis=1)
  side = jnp.concatenate(
      [s0_ref[...], s1_ref[...], s2_ref[...], s3_ref[...]], axis=1)
  sum_e = jnp.dot(side, wgc_ref[...], precision=lax.Precision.HIGHEST,
                  preferred_element_type=jnp.float32) + bgc_ref[...]
  bi = jnp.dot(ego * side, wbi_ref[...], precision=lax.Precision.HIGHEST,
               preferred_element_type=jnp.float32) + bbi_ref[...]
  e = sum_e + bi
  e = jnp.where(e >= 0.0, e, 0.2 * e)
  nrm = jnp.maximum(jnp.sqrt(jnp.sum(e * e, axis=1, keepdims=True)), 1e-12)
  n = e / nrm
  if is_last:
    outs[0][...] = (accin_ref[...] + n) * 0.25
  else:
    for q in range(4):
      outs[q][...] = e[:, q * DQ:(q + 1) * DQ]
    outs[4][...] = accin_ref[...] + n


def _make_hop(is_last):
  nblk = N_PAD // ROW_BLK
  row = lambda i: (i, 0)
  full = lambda i: (0, 0)
  in_specs = (
      [pl.BlockSpec((ROW_BLK, DQ), row) for _ in range(8)]
      + [
          pl.BlockSpec((D, D), full),
          pl.BlockSpec((1, D), full),
          pl.BlockSpec((D, D), full),
          pl.BlockSpec((1, D), full),
          pl.BlockSpec((ROW_BLK, D), row),
      ]
  )
  if is_last:
    out_specs = [pl.BlockSpec((ROW_BLK, D), row)]
    out_shape = [jax.ShapeDtypeStruct((N_PAD, D), jnp.float32)]
  else:
    out_specs = (
        [pl.BlockSpec((ROW_BLK, DQ), row) for _ in range(4)]
        + [pl.BlockSpec((ROW_BLK, D), row)]
    )
    out_shape = (
        [jax.ShapeDtypeStruct((N_PAD, DQ), jnp.float32) for _ in range(4)]
        + [jax.ShapeDtypeStruct((N_PAD, D), jnp.float32)]
    )
  return pl.pallas_call(
      functools.partial(_hop_body, is_last),
      grid=(nblk,),
      in_specs=in_specs,
      out_specs=out_specs,
      out_shape=out_shape,
  )


_hop_mid = _make_hop(False)
_hop_last = _make_hop(True)
HOPS_LAST = 2


@jax.jit
def kernel(user_emb, item_emb, adj_idx, adj_val,
           W_gc_0, b_gc_0, W_bi_0, b_bi_0,
           W_gc_1, b_gc_1, W_bi_1, b_bi_1,
           W_gc_2, b_gc_2, W_bi_2, b_bi_2):
  ego0 = jnp.concatenate([user_emb, item_emb], axis=0)
  ego0 = jnp.pad(ego0, ((0, N_PAD - N), (0, 0)))
  eq = [ego0[:, q * DQ:(q + 1) * DQ] for q in range(4)]

  row = adj_idx[0].astype(jnp.int32)
  col = adj_idx[1].astype(jnp.int32)
  rowp = jnp.pad(row, (0, NNZ_PAD - NNZ)).reshape(NNZ_PAD // GRP, GRP)
  colp = jnp.pad(col, (0, NNZ_PAD - NNZ)).reshape(NNZ_PAD // GRP, GRP)
  valp = jnp.pad(adj_val, (0, NNZ_PAD - NNZ))
  zrows = jnp.zeros((N_PAD, DQ), jnp.float32)

  weights = [(W_gc_0, b_gc_0, W_bi_0, b_bi_0),
             (W_gc_1, b_gc_1, W_bi_1, b_bi_1),
             (W_gc_2, b_gc_2, W_bi_2, b_bi_2)]

  acc = ego0
  for k, (wgc, bgc, wbi, bbi) in enumerate(weights):
    sq = _spmm(eq[0], eq[1], eq[2], eq[3], rowp, colp, valp, zrows)
    if k < HOPS_LAST:
      *eq, acc = _hop_mid(*eq, *sq, wgc, bgc, wbi, bbi, acc)
    else:
      final, = _hop_last(*eq, *sq, wgc, bgc, wbi, bbi, acc)
  return final[:N]
